# Initial kernel scaffold; baseline (speedup 1.0000x reference)
import jax
import jax.numpy as jnp
from jax.experimental import pallas as pl


def kernel(x, bias):
    raise NotImplementedError("write your pallas kernel here")


# speedup vs baseline: 1.1371x; 1.1371x over previous
import functools

import jax
import jax.numpy as jnp
from jax.experimental import pallas as pl
from jax.experimental.pallas import tpu as pltpu


def _pool_kernel(x_ref, bias_ref, o_ref, *, inv_hw):
    b = bias_ref[...]
    n_slices = x_ref.shape[1] // 128

    acc0 = jnp.where(x_ref[:, 0:128] > b, 1.0, 0.0)
    acc1 = jnp.where(x_ref[:, 128:256] > b, 1.0, 0.0)
    for j in range(2, n_slices):
        g = jnp.where(x_ref[:, j * 128:(j + 1) * 128] > b, 1.0, 0.0)
        if j % 2 == 0:
            acc0 = acc0 + g
        else:
            acc1 = acc1 + g

    o_ref[...] = jnp.sum(acc0 + acc1, axis=-1, keepdims=True) * inv_hw


def kernel(x, bias):
    B, C, H, W = x.shape
    BC, HW = B * C, H * W
    assert HW % 128 == 0

    x2 = x.reshape(BC, HW)
    bias2 = jnp.tile(bias.astype(x.dtype), B).reshape(BC, 1)

    TM = 128
    grid = pl.cdiv(BC, TM)

    out2 = pl.pallas_call(
        functools.partial(_pool_kernel, inv_hw=1.0 / HW),
        out_shape=jax.ShapeDtypeStruct((BC, 1), jnp.float32),
        grid=(grid,),
        in_specs=[
            pl.BlockSpec((TM, HW), lambda i: (i, 0)),
            pl.BlockSpec((TM, 1), lambda i: (i, 0)),
        ],
        out_specs=pl.BlockSpec((TM, 1), lambda i: (i, 0)),
        compiler_params=pltpu.CompilerParams(
            dimension_semantics=("parallel",),
        ),
    )(x2, bias2)

    return out2.reshape(B, C, 1, 1)
